# Initial kernel scaffold; baseline (speedup 1.0000x reference)
import jax
import jax.numpy as jnp
from jax.experimental import pallas as pl


def kernel(x_enc, N_matrix, Q_init, line1_w, line1_b, line2_w, line2_b, lin1_w, lin1_b, lin2_w, lin2_b):
    raise NotImplementedError("write your pallas kernel here")


# speedup vs baseline: 58.4661x; 58.4661x over previous
import functools

import jax
import jax.numpy as jnp
import numpy as np
from jax.experimental import pallas as pl
from jax.experimental.pallas import tpu as pltpu

_N = 5000
_D = 256
_ALPHA = 3.0
_K = 2500
_STEPS = 10
_LR = 0.1
_BQ = 48
_TR = 200
_BR = 200
_NP = 5120
_BC = 1280
_HI_BITS = 0x3F82A000


def _dot(a, b, dims):
    return jax.lax.dot_general(
        a, b, (dims, ((), ())),
        preferred_element_type=jnp.float32)


_C1 = tuple(np.float32(s) for s in (
    "0.1", "0.526315808", "0.369003713", "0.290782213", "0.244194299",
    "0.213420302", "0.191679895", "0.175582513", "0.163244113", "0.153534"))
_C2 = tuple(np.float32(s) for s in (
    "999.999939", "500.250092", "333.666901", "250.37532", "200.400391",
    "167.083817", "143.286285", "125.438156", "111.556297", "100.450829"))
_B1M = np.float32("0.0899999961")
_F01 = np.float32("0.1")
_F09 = np.float32("0.9")
_F999 = np.float32("0.999")
_F001 = np.float32("0.001")
_EPS = np.float32("1e-08")


def _adam_mv(t, g, m_prev, v_prev):
    if t == 1:
        m = g
        v = (_F001 * g) * g
    elif t == 2:
        m = _B1M * m_prev + _F01 * g
        v = _F999 * v_prev + (_F001 * g) * g
    else:
        m = _F09 * m_prev + _F01 * g
        v = _F999 * v_prev + (_F001 * g) * g
    return m, v


def _adam_delta(t, m, v):
    if t == 1:
        num = _F01 * m
    else:
        num = m * _C1[t - 1]
    return num / (jnp.sqrt(v * _C2[t - 1]) + _EPS)


def _prep_kernel(nmat_ref, qall_ref, r2_ref,
                 l1w_ref, l1b_ref, l2w_ref, l2b_ref,
                 lin1w_ref, lin1b_ref, lin2w_ref, lin2b_ref,
                 nv1_ref, nv2_ref,
                 p_scr, mp_scr, vp_scr, e_scr):
    n_tiles = _N // _TR

    def init_tile(r, _):
        idx = pl.ds(r * _TR, _TR)
        p_scr[idx, :] = nmat_ref[idx, :]
        return 0

    jax.lax.fori_loop(0, n_tiles, init_tile, 0)

    Q = qall_ref[...]
    mQ = None
    vQ = None
    for t in range(1, _STEPS + 1):
        def e_tile(r, _, Q=Q):
            idx = pl.ds(r * _TR, _TR)
            pred = _dot(p_scr[idx, :], Q, ((1,), (1,)))
            e_scr[idx, :] = 2.0 * (pred - r2_ref[idx, :])
            return 0

        jax.lax.fori_loop(0, n_tiles, e_tile, 0)
        gQ = _dot(e_scr[...], p_scr[...], ((0,), (0,)))

        def p_tile(r, _, Q=Q, t=t):
            idx = pl.ds(r * _TR, _TR)
            g = _dot(e_scr[idx, :], Q, ((1,), (0,)))
            m_prev = mp_scr[idx, :] if t > 1 else None
            v_prev = vp_scr[idx, :] if t > 1 else None
            m, v = _adam_mv(t, g, m_prev, v_prev)
            mp_scr[idx, :] = m
            vp_scr[idx, :] = v
            p_scr[idx, :] = p_scr[idx, :] - _adam_delta(t, m, v)
            return 0

        jax.lax.fori_loop(0, n_tiles, p_tile, 0)
        mQ, vQ = _adam_mv(t, gQ, mQ, vQ)
        Q = Q - _adam_delta(t, mQ, vQ)

    T_m = (Q[11:12, :] + Q[23:24, :] + Q[35:36, :] + Q[47:48, :]) * 0.25
    t_lin = _dot(T_m, l2w_ref[...], ((1,), (1,))) + l2b_ref[...]
    l1w = l1w_ref[...]
    l1b = l1b_ref[...]
    lin1w = lin1w_ref[...]
    lin1b = lin1b_ref[...]
    lin2w = lin2w_ref[...]
    lin2b = lin2b_ref[...]

    def out_tile(r, _):
        idx = pl.ds(r * _TR, _TR)
        P_t = p_scr[idx, :]
        nv = jnp.maximum(_dot(P_t, l1w, ((1,), (1,))) + l1b + t_lin, 0.0)
        nv1_ref[idx, :] = jnp.tanh(
            _ALPHA * (_dot(nv, lin1w, ((1,), (1,))) + lin1b))
        nv2_ref[idx, :] = jnp.tanh(
            _ALPHA * (_dot(nv, lin2w, ((1,), (1,))) + lin2b))
        return 0

    jax.lax.fori_loop(0, n_tiles, out_tile, 0)


def _select_kernel(nv1b_ref, nv2b_ref, nv1f_ref, nv2f_ref, noise_ref,
                   out_ref, adj_scr, sb_scr):
    nv1b = nv1b_ref[...]
    nv2b = nv2b_ref[...]
    n_ctiles = _NP // _BC

    def col_body(c, _):
        idx = pl.ds(c * _BC, _BC)
        M1 = _dot(nv1b, nv2f_ref[idx, :], ((1,), (1,)))
        M2 = _dot(nv2b, nv1f_ref[idx, :], ((1,), (1,)))
        adj = jnp.maximum(jnp.tanh(_ALPHA * (M1 - M2)), 0.0)
        s = adj + noise_ref[:, idx]
        adj_scr[:, idx] = adj
        sb_scr[:, idx] = jax.lax.bitcast_convert_type(s, jnp.int32)
        return 0

    jax.lax.fori_loop(0, n_ctiles, col_body, 0)

    lo = jnp.zeros((_BR, 1), jnp.int32)
    hi = jnp.full((_BR, 1), _HI_BITS, jnp.int32)

    def body(_, carry):
        lo, hi = carry
        mid = lo + (hi - lo) // 2
        cnt = jnp.sum((sb_scr[...] >= mid).astype(jnp.float32), axis=1,
                      keepdims=True)
        ge = cnt >= float(_K)
        return jnp.where(ge, mid, lo), jnp.where(ge, hi, mid)

    lo, hi = jax.lax.fori_loop(0, 31, body, (lo, hi))

    c_gt = jnp.sum((sb_scr[...] > lo).astype(jnp.float32), axis=1,
                   keepdims=True)
    iota = jax.lax.broadcasted_iota(jnp.int32, (_BR, _NP), 1)

    lo2 = jnp.zeros((_BR, 1), jnp.int32)
    hi2 = jnp.full((_BR, 1), _N, jnp.int32)

    def body2(_, carry):
        lo2, hi2 = carry
        mid = lo2 + (hi2 - lo2) // 2
        sb = sb_scr[...]
        cnt = c_gt + jnp.sum(((sb == lo) & (iota < mid)).astype(jnp.float32),
                             axis=1, keepdims=True)
        ok = cnt >= float(_K)
        return jnp.where(ok, lo2, mid), jnp.where(ok, mid, hi2)

    lo2, hi2 = jax.lax.fori_loop(0, 13, body2, (lo2, hi2))

    sb = sb_scr[...]
    keep = (sb > lo) | ((sb == lo) & (iota < hi2))
    out_ref[...] = jnp.where(keep, adj_scr[...], 0.0)


@functools.partial(jax.jit, static_argnums=())
def _run(x_enc, N_matrix, Q_init, line1_w, line1_b, line2_w, line2_b,
         lin1_w, lin1_b, lin2_w, lin2_b):
    R2 = jnp.transpose(x_enc[:, :, :, 0], (1, 0, 2)).reshape(_N, _BQ)
    Qall = Q_init.reshape(_BQ, _D)
    nv1, nv2 = pl.pallas_call(
        _prep_kernel,
        out_shape=[jax.ShapeDtypeStruct((_N, _D), jnp.float32),
                   jax.ShapeDtypeStruct((_N, _D), jnp.float32)],
        scratch_shapes=[
            pltpu.VMEM((_N, _D), jnp.float32),
            pltpu.VMEM((_N, _D), jnp.float32),
            pltpu.VMEM((_N, _D), jnp.float32),
            pltpu.VMEM((_N, _BQ), jnp.float32),
        ],
        compiler_params=pltpu.CompilerParams(
            vmem_limit_bytes=60 * 1024 * 1024),
    )(N_matrix, Qall, R2,
      line1_w, line1_b.reshape(1, _D), line2_w, line2_b.reshape(1, _D),
      lin1_w, lin1_b.reshape(1, _D), lin2_w, lin2_b.reshape(1, _D))

    noise = jax.random.uniform(jax.random.key(1234), (_N, _N),
                               dtype=jnp.float32) * 0.01
    noise_p = jnp.pad(noise, ((0, 0), (0, _NP - _N)))
    nv1_p = jnp.pad(nv1, ((0, _NP - _N), (0, 0)))
    nv2_p = jnp.pad(nv2, ((0, _NP - _N), (0, 0)))
    grid = _N // _BR
    out = pl.pallas_call(
        _select_kernel,
        grid=(grid,),
        in_specs=[
            pl.BlockSpec((_BR, _D), lambda i: (i, 0)),
            pl.BlockSpec((_BR, _D), lambda i: (i, 0)),
            pl.BlockSpec((_NP, _D), lambda i: (0, 0)),
            pl.BlockSpec((_NP, _D), lambda i: (0, 0)),
            pl.BlockSpec((_BR, _NP), lambda i: (i, 0)),
        ],
        out_specs=pl.BlockSpec((_BR, _NP), lambda i: (i, 0)),
        out_shape=jax.ShapeDtypeStruct((_N, _NP), jnp.float32),
        scratch_shapes=[
            pltpu.VMEM((_BR, _NP), jnp.float32),
            pltpu.VMEM((_BR, _NP), jnp.int32),
        ],
        compiler_params=pltpu.CompilerParams(
            vmem_limit_bytes=60 * 1024 * 1024),
    )(nv1, nv2, nv1_p, nv2_p, noise_p)
    return out[:, :_N]


def kernel(x_enc, N_matrix, Q_init, line1_w, line1_b, line2_w, line2_b,
           lin1_w, lin1_b, lin2_w, lin2_b):
    return _run(x_enc, N_matrix, Q_init, line1_w, line1_b, line2_w, line2_b,
                lin1_w, lin1_b, lin2_w, lin2_b)


# speedup vs baseline: 62.2569x; 1.0648x over previous
import functools

import jax
import jax.numpy as jnp
import numpy as np
from jax.experimental import pallas as pl
from jax.experimental.pallas import tpu as pltpu

_N = 5000
_D = 256
_ALPHA = 3.0
_K = 2500
_STEPS = 10
_LR = 0.1
_BQ = 48
_TR = 200
_BR = 200
_NP = 5120
_BC = 1280
_HI_BITS = 0x3F82A000


def _dot(a, b, dims):
    return jax.lax.dot_general(
        a, b, (dims, ((), ())),
        preferred_element_type=jnp.float32)


_C1 = tuple(np.float32(s) for s in (
    "0.1", "0.526315808", "0.369003713", "0.290782213", "0.244194299",
    "0.213420302", "0.191679895", "0.175582513", "0.163244113", "0.153534"))
_C2 = tuple(np.float32(s) for s in (
    "999.999939", "500.250092", "333.666901", "250.37532", "200.400391",
    "167.083817", "143.286285", "125.438156", "111.556297", "100.450829"))
_B1M = np.float32("0.0899999961")
_F01 = np.float32("0.1")
_F09 = np.float32("0.9")
_F999 = np.float32("0.999")
_F001 = np.float32("0.001")
_EPS = np.float32("1e-08")


def _adam_mv(t, g, m_prev, v_prev):
    if t == 1:
        m = g
        v = (_F001 * g) * g
    elif t == 2:
        m = _B1M * m_prev + _F01 * g
        v = _F999 * v_prev + (_F001 * g) * g
    else:
        m = _F09 * m_prev + _F01 * g
        v = _F999 * v_prev + (_F001 * g) * g
    return m, v


def _adam_delta(t, m, v):
    if t == 1:
        num = _F01 * m
    else:
        num = m * _C1[t - 1]
    return num / (jnp.sqrt(v * _C2[t - 1]) + _EPS)


def _prep_kernel(nmat_ref, qall_ref, r2_ref,
                 l1w_ref, l1b_ref, l2w_ref, l2b_ref,
                 lin1w_ref, lin1b_ref, lin2w_ref, lin2b_ref,
                 nv1_ref, nv2_ref,
                 p_scr, mp_scr, vp_scr, e_scr):
    n_tiles = _N // _TR

    def init_tile(r, _):
        idx = pl.ds(r * _TR, _TR)
        p_scr[idx, :] = nmat_ref[idx, :]
        return 0

    jax.lax.fori_loop(0, n_tiles, init_tile, 0)

    Q = qall_ref[...]
    mQ = None
    vQ = None
    for t in range(1, _STEPS + 1):
        def e_tile(r, _, Q=Q):
            idx = pl.ds(r * _TR, _TR)
            pred = _dot(p_scr[idx, :], Q, ((1,), (1,)))
            e_scr[idx, :] = 2.0 * (pred - r2_ref[idx, :])
            return 0

        jax.lax.fori_loop(0, n_tiles, e_tile, 0)
        gQ = _dot(e_scr[...], p_scr[...], ((0,), (0,)))

        def p_tile(r, _, Q=Q, t=t):
            idx = pl.ds(r * _TR, _TR)
            g = _dot(e_scr[idx, :], Q, ((1,), (0,)))
            m_prev = mp_scr[idx, :] if t > 1 else None
            v_prev = vp_scr[idx, :] if t > 1 else None
            m, v = _adam_mv(t, g, m_prev, v_prev)
            mp_scr[idx, :] = m
            vp_scr[idx, :] = v
            p_scr[idx, :] = p_scr[idx, :] - _adam_delta(t, m, v)
            return 0

        jax.lax.fori_loop(0, n_tiles, p_tile, 0)
        mQ, vQ = _adam_mv(t, gQ, mQ, vQ)
        Q = Q - _adam_delta(t, mQ, vQ)

    T_m = (Q[11:12, :] + Q[23:24, :] + Q[35:36, :] + Q[47:48, :]) * 0.25
    t_lin = _dot(T_m, l2w_ref[...], ((1,), (1,))) + l2b_ref[...]
    l1w = l1w_ref[...]
    l1b = l1b_ref[...]
    lin1w = lin1w_ref[...]
    lin1b = lin1b_ref[...]
    lin2w = lin2w_ref[...]
    lin2b = lin2b_ref[...]

    def out_tile(r, _):
        idx = pl.ds(r * _TR, _TR)
        P_t = p_scr[idx, :]
        nv = jnp.maximum(_dot(P_t, l1w, ((1,), (1,))) + l1b + t_lin, 0.0)
        nv1_ref[idx, :] = jnp.tanh(
            _ALPHA * (_dot(nv, lin1w, ((1,), (1,))) + lin1b))
        nv2_ref[idx, :] = jnp.tanh(
            _ALPHA * (_dot(nv, lin2w, ((1,), (1,))) + lin2b))
        return 0

    jax.lax.fori_loop(0, n_tiles, out_tile, 0)


def _select_kernel(nv1b_ref, nv2b_ref, nv1f_ref, nv2f_ref, noise_ref,
                   out_ref, adj_scr, sb_scr):
    nv1b = nv1b_ref[...]
    nv2b = nv2b_ref[...]

    for off in range(0, _N, _BC):
        w = min(_BC, _N - off)
        idx = pl.ds(off, w)
        M1 = _dot(nv1b, nv2f_ref[idx, :], ((1,), (1,)))
        M2 = _dot(nv2b, nv1f_ref[idx, :], ((1,), (1,)))
        adj = jnp.maximum(jnp.tanh(_ALPHA * (M1 - M2)), 0.0)
        s = adj + noise_ref[:, idx]
        adj_scr[:, idx] = adj
        sb_scr[:, idx] = jax.lax.bitcast_convert_type(s, jnp.int32)

    lo = jnp.zeros((_BR, 1), jnp.int32)
    hi = jnp.full((_BR, 1), _HI_BITS, jnp.int32)

    def body(_, carry):
        lo, hi = carry
        mid = lo + (hi - lo) // 2
        cnt = jnp.sum((sb_scr[...] >= mid).astype(jnp.float32), axis=1,
                      keepdims=True)
        ge = cnt >= float(_K)
        return jnp.where(ge, mid, lo), jnp.where(ge, hi, mid)

    lo, hi = jax.lax.fori_loop(0, 31, body, (lo, hi))

    c_gt = jnp.sum((sb_scr[...] > lo).astype(jnp.float32), axis=1,
                   keepdims=True)
    iota = jax.lax.broadcasted_iota(jnp.int32, (_BR, _N), 1)

    lo2 = jnp.zeros((_BR, 1), jnp.int32)
    hi2 = jnp.full((_BR, 1), _N, jnp.int32)

    def body2(_, carry):
        lo2, hi2 = carry
        mid = lo2 + (hi2 - lo2) // 2
        sb = sb_scr[...]
        cnt = c_gt + jnp.sum(((sb == lo) & (iota < mid)).astype(jnp.float32),
                             axis=1, keepdims=True)
        ok = cnt >= float(_K)
        return jnp.where(ok, lo2, mid), jnp.where(ok, mid, hi2)

    lo2, hi2 = jax.lax.fori_loop(0, 13, body2, (lo2, hi2))

    sb = sb_scr[...]
    keep = (sb > lo) | ((sb == lo) & (iota < hi2))
    out_ref[...] = jnp.where(keep, adj_scr[...], 0.0)


@functools.partial(jax.jit, static_argnums=())
def _run(x_enc, N_matrix, Q_init, line1_w, line1_b, line2_w, line2_b,
         lin1_w, lin1_b, lin2_w, lin2_b):
    R2 = jnp.transpose(x_enc[:, :, :, 0], (1, 0, 2)).reshape(_N, _BQ)
    Qall = Q_init.reshape(_BQ, _D)
    nv1, nv2 = pl.pallas_call(
        _prep_kernel,
        out_shape=[jax.ShapeDtypeStruct((_N, _D), jnp.float32),
                   jax.ShapeDtypeStruct((_N, _D), jnp.float32)],
        scratch_shapes=[
            pltpu.VMEM((_N, _D), jnp.float32),
            pltpu.VMEM((_N, _D), jnp.float32),
            pltpu.VMEM((_N, _D), jnp.float32),
            pltpu.VMEM((_N, _BQ), jnp.float32),
        ],
        compiler_params=pltpu.CompilerParams(
            vmem_limit_bytes=60 * 1024 * 1024),
    )(N_matrix, Qall, R2,
      line1_w, line1_b.reshape(1, _D), line2_w, line2_b.reshape(1, _D),
      lin1_w, lin1_b.reshape(1, _D), lin2_w, lin2_b.reshape(1, _D))

    noise = jax.random.uniform(jax.random.key(1234), (_N, _N),
                               dtype=jnp.float32) * 0.01
    grid = _N // _BR
    out = pl.pallas_call(
        _select_kernel,
        grid=(grid,),
        in_specs=[
            pl.BlockSpec((_BR, _D), lambda i: (i, 0)),
            pl.BlockSpec((_BR, _D), lambda i: (i, 0)),
            pl.BlockSpec((_N, _D), lambda i: (0, 0)),
            pl.BlockSpec((_N, _D), lambda i: (0, 0)),
            pl.BlockSpec((_BR, _N), lambda i: (i, 0)),
        ],
        out_specs=pl.BlockSpec((_BR, _N), lambda i: (i, 0)),
        out_shape=jax.ShapeDtypeStruct((_N, _N), jnp.float32),
        scratch_shapes=[
            pltpu.VMEM((_BR, _N), jnp.float32),
            pltpu.VMEM((_BR, _N), jnp.int32),
        ],
        compiler_params=pltpu.CompilerParams(
            vmem_limit_bytes=60 * 1024 * 1024),
    )(nv1, nv2, nv1, nv2, noise)
    return out


def kernel(x_enc, N_matrix, Q_init, line1_w, line1_b, line2_w, line2_b,
           lin1_w, lin1_b, lin2_w, lin2_b):
    return _run(x_enc, N_matrix, Q_init, line1_w, line1_b, line2_w, line2_b,
                lin1_w, lin1_b, lin2_w, lin2_b)
